# baseline (speedup 1.0000x reference)
import functools

import jax
import jax.numpy as jnp
from jax import lax
from jax.experimental import pallas as pl
from jax.experimental.pallas import tpu as pltpu
from jax.experimental.pallas import tpu_sc as plsc

NUM_CAT = 26
NUM_CONT = 13
V = 100000
D = 16
B = 16384
H1 = 64
H2 = 32
N = B * NUM_CAT


def _make_sc_gather():
    NC, NS, L = 2, 16, 16
    NW = NC * NS
    PW = N // NW
    SUB = 8
    G = 128
    ROWS = SUB * G
    MC = PW // ROWS
    PERIOD = 13

    mesh = plsc.VectorSubcoreMesh(
        core_axis_name="c", subcore_axis_name="s",
        num_cores=NC, num_subcores=NS)

    @functools.partial(
        pl.kernel,
        mesh=mesh,
        out_type=(
            jax.ShapeDtypeStruct((N, D), jnp.float32),
            jax.ShapeDtypeStruct((N,), jnp.float32),
        ),
        scratch_types=[
            pltpu.VMEM((PW,), jnp.int32),
            pltpu.VMEM((ROWS, D), jnp.float32),
            pltpu.VMEM((ROWS,), jnp.float32),
            pltpu.SemaphoreType.DMA,
            pltpu.SemaphoreType.DMA,
        ],
        compiler_params=pltpu.CompilerParams(use_tc_tiling_on_sc=False),
    )
    def sc_gather(idx_hbm, tab_hbm, lin_hbm, emb_out, lin_out,
                  idxv, rowsv, linv, sem_e, sem_l):
        wid = lax.axis_index("s") * NC + lax.axis_index("c")
        base = wid * PW
        pltpu.sync_copy(idx_hbm.at[pl.ds(base, PW)], idxv)

        iota = lax.iota(jnp.int32, L)
        offs = [((iota + r * L) % NUM_CAT) * V for r in range(PERIOD)]

        def add_offsets(g, carry):
            for r in range(PERIOD):
                p = g * (PERIOD * L) + r * L
                idxv[pl.ds(p, L)] = idxv[pl.ds(p, L)] + offs[r]
            return carry

        lax.fori_loop(0, PW // (PERIOD * L), add_offsets, 0)

        def chunk(m, carry):
            cps = []
            for s in range(SUB):
                isl = idxv.at[pl.ds(m * ROWS + s * G, G)]
                cps.append(pltpu.async_copy(
                    tab_hbm.at[isl], rowsv.at[pl.ds(s * G, G)], sem_e))
                cps.append(pltpu.async_copy(
                    lin_hbm.at[isl], linv.at[pl.ds(s * G, G)], sem_l))
            for cp in cps:
                cp.wait()
            pltpu.sync_copy(rowsv, emb_out.at[pl.ds(base + m * ROWS, ROWS)])
            pltpu.sync_copy(linv, lin_out.at[pl.ds(base + m * ROWS, ROWS)])
            return carry

        lax.fori_loop(0, MC, chunk, 0)

    return sc_gather


@functools.lru_cache(maxsize=1)
def _get_sc_gather():
    return _make_sc_gather()


_TILE = 1024
_CD = NUM_CAT * D
_KD = NUM_CONT * D


def _tc_a_body(ce_ref, cont_ref, catl_ref, w1_ref, b1_ref, cw_ref, cwf_ref,
               clw_ref, clb_ref, fmb_ref, h1_ref, s_ref):
    ce = ce_ref[...]
    cont = cont_ref[...]
    cw = cw_ref[...]

    rows = lax.broadcasted_iota(jnp.int32, (_CD, D), 0)
    cols = lax.broadcasted_iota(jnp.int32, (_CD, D), 1)
    S = (rows % D == cols).astype(jnp.float32)

    sum_emb = jnp.dot(ce, S, preferred_element_type=jnp.float32) \
        + jnp.dot(cont, cw, preferred_element_type=jnp.float32)
    sum_sq = jnp.dot(ce * ce, S, preferred_element_type=jnp.float32) \
        + jnp.dot(cont * cont, cw * cw, preferred_element_type=jnp.float32)
    fm = 0.5 * jnp.sum(sum_emb * sum_emb - sum_sq, axis=1, keepdims=True)

    lin = jnp.sum(catl_ref[...], axis=1, keepdims=True) \
        + jnp.sum(cont * clw_ref[...] + clb_ref[...], axis=1, keepdims=True) \
        + fmb_ref[...]
    s_ref[...] = lin + fm

    W1 = w1_ref[...]
    W1cat = W1[:_CD, :]
    W1cont = W1[_CD:, :]
    grows = lax.broadcasted_iota(jnp.int32, (NUM_CONT, _KD), 0)
    gcols = lax.broadcasted_iota(jnp.int32, (NUM_CONT, _KD), 1)
    Gm = (gcols // D == grows).astype(jnp.float32)
    Wc = jnp.dot(Gm, cwf_ref[...] * W1cont, preferred_element_type=jnp.float32)
    h1 = jnp.dot(ce, W1cat, preferred_element_type=jnp.float32) \
        + jnp.dot(cont, Wc, preferred_element_type=jnp.float32) + b1_ref[...]
    h1_ref[...] = h1


_tc_a = pl.pallas_call(
    _tc_a_body,
    grid=(B // _TILE,),
    in_specs=[
        pl.BlockSpec((_TILE, _CD), lambda i: (i, 0)),
        pl.BlockSpec((_TILE, NUM_CONT), lambda i: (i, 0)),
        pl.BlockSpec((_TILE, NUM_CAT), lambda i: (i, 0)),
        pl.BlockSpec(((NUM_CAT + NUM_CONT) * D, H1), lambda i: (0, 0)),
        pl.BlockSpec((1, H1), lambda i: (0, 0)),
        pl.BlockSpec((NUM_CONT, D), lambda i: (0, 0)),
        pl.BlockSpec((_KD, 1), lambda i: (0, 0)),
        pl.BlockSpec((1, NUM_CONT), lambda i: (0, 0)),
        pl.BlockSpec((1, NUM_CONT), lambda i: (0, 0)),
        pl.BlockSpec((1, 1), lambda i: (0, 0)),
    ],
    out_specs=[
        pl.BlockSpec((_TILE, H1), lambda i: (i, 0)),
        pl.BlockSpec((_TILE, 1), lambda i: (i, 0)),
    ],
    out_shape=[
        jax.ShapeDtypeStruct((B, H1), jnp.float32),
        jax.ShapeDtypeStruct((B, 1), jnp.float32),
    ],
    compiler_params=pltpu.CompilerParams(
        dimension_semantics=("parallel",)),
)


def _bn_kernel(h, g, b):
    m = jnp.mean(h, axis=0, keepdims=True)
    v = jnp.mean((h - m) ** 2, axis=0, keepdims=True)
    return (h - m) / jnp.sqrt(v + 1e-5) * g + b


def _tc_b_body(h1_ref, s_ref, g1_ref, be1_ref, w2_ref, b2_ref, g2_ref,
               be2_ref, w3_ref, b3_ref, out_ref):
    h = jnp.maximum(_bn_kernel(h1_ref[...], g1_ref[...], be1_ref[...]), 0.0)
    h = jnp.dot(h, w2_ref[...], preferred_element_type=jnp.float32) + b2_ref[...]
    h = jnp.maximum(_bn_kernel(h, g2_ref[...], be2_ref[...]), 0.0)
    dnn = jnp.dot(h, w3_ref[...], preferred_element_type=jnp.float32) + b3_ref[...]
    z = s_ref[...] + dnn
    out_ref[...] = 1.0 / (1.0 + jnp.exp(-z))


_tc_b = pl.pallas_call(
    _tc_b_body,
    out_shape=jax.ShapeDtypeStruct((B, 1), jnp.float32),
)


def kernel(x, cat_tables, cont_w, cat_lin, cont_lin_w, cont_lin_b, fm_bias,
           W1, b1, g1, be1, W2, b2, g2, be2, W3, b3):
    idx = x[:, :NUM_CAT].astype(jnp.int32).reshape(-1)
    cont = x[:, NUM_CAT:]
    tab = cat_tables.reshape(NUM_CAT * V, D)
    lint = cat_lin.reshape(NUM_CAT * V)

    emb_flat, catl_flat = _get_sc_gather()(idx, tab, lint)
    ce = emb_flat.reshape(B, _CD)
    catl = catl_flat.reshape(B, NUM_CAT)

    h1, s = _tc_a(ce, cont, catl, W1, b1.reshape(1, H1), cont_w,
                  cont_w.reshape(_KD, 1), cont_lin_w.reshape(1, NUM_CONT),
                  cont_lin_b.reshape(1, NUM_CONT), fm_bias.reshape(1, 1))
    return _tc_b(h1, s, g1.reshape(1, H1), be1.reshape(1, H1), W2,
                 b2.reshape(1, H2), g2.reshape(1, H2), be2.reshape(1, H2),
                 W3, b3.reshape(1, 1))
